# baseline (speedup 1.0000x reference)
import jax
import jax.numpy as jnp
from jax import lax
from jax.experimental import pallas as pl
from jax.experimental.pallas import tpu as pltpu
from jax.experimental.pallas import tpu_sc as plsc

_N = 8192
_K = 8192
_D = 256
_NB = 4096
_KB = 8192
_GN = _N // _NB
_GK = _K // _KB


def _argmin_body(z_ref, e_ref, idx_ref, zneg_ref,
                 nrm_s, zbf_s, ebf_s):
    n = pl.program_id(0)
    k = pl.program_id(1)

    @pl.when(k == 0)
    def _():
        zf = z_ref[...]
        zbf_s[...] = zf.astype(jnp.bfloat16)
        zneg_ref[...] = -zf

    @pl.when(n == 0)
    def _():
        ef = e_ref[...]
        ebf_s[k] = ef.astype(jnp.bfloat16)
        nrm_s[k] = jnp.reshape(1.0 - 0.5 * jnp.sum(ef * ef, axis=1),
                               (_KB // 128, 128))

    nrm_rows = nrm_s[k]
    zbf = zbf_s[...]
    bv = None
    for g in range(_KB // 256):
        sg = lax.dot_general(
            zbf, ebf_s[k, g * 256:(g + 1) * 256, :],
            (((1,), (1,)), ((), ())),
            preferred_element_type=jnp.float32)
        for jj in range(2):
            j = 2 * g + jj
            sl = slice(jj * 128, (jj + 1) * 128)
            sj = sg[:, sl] + nrm_rows[j:j + 1, :]
            kj = ((lax.bitcast_convert_type(sj, jnp.int32) & jnp.int32(-64))
                  | (_KB // 128 - 1 - j))
            kjf = lax.bitcast_convert_type(kj, jnp.float32)
            bv = kjf if bv is None else jnp.maximum(bv, kjf)

    kmax = jnp.max(bv, axis=1, keepdims=True)
    lane_iota = lax.broadcasted_iota(jnp.int32, (_NB, 128), 1)
    lane = jnp.min(jnp.where(bv == kmax, lane_iota, jnp.int32(2 ** 30)),
                   axis=1, keepdims=True)
    id6 = lax.bitcast_convert_type(kmax, jnp.int32) & 63
    strip = _KB // 128 - 1 - id6
    b = strip * 128 + lane
    idx_ref[...] = jnp.reshape(b, (_NB,))


_argmin_call = pl.pallas_call(
    _argmin_body,
    grid=(_GN, _GK),
    in_specs=[
        pl.BlockSpec((_NB, _D), lambda n, k: (n, 0)),
        pl.BlockSpec((_KB, _D), lambda n, k: (jnp.where(n == 0, k, _GK - 1), 0)),
    ],
    out_specs=[
        pl.BlockSpec((_NB,), lambda n, k: (n,)),
        pl.BlockSpec((_NB, _D), lambda n, k: (n, 0)),
    ],
    out_shape=[
        jax.ShapeDtypeStruct((_N,), jnp.int32),
        jax.ShapeDtypeStruct((_N, _D), jnp.float32),
    ],
    scratch_shapes=[
        pltpu.VMEM((_GK, _KB // 128, 128), jnp.float32),
        pltpu.VMEM((_NB, _D), jnp.bfloat16),
        pltpu.VMEM((_GK, _KB, _D), jnp.bfloat16),
    ],
    compiler_params=pltpu.CompilerParams(
        dimension_semantics=("arbitrary", "arbitrary")),
)

_CH = 64
_RPW = _N // 32
_NCH = _RPW // _CH


def _gather_combine(emb, idx, zneg):
    mesh = plsc.VectorSubcoreMesh(core_axis_name="c", subcore_axis_name="s")

    @pl.kernel(out_type=jax.ShapeDtypeStruct((_N, _D), jnp.float32),
               mesh=mesh,
               scratch_types=[
                   pltpu.VMEM((_RPW,), jnp.int32),
                   pltpu.VMEM((2, _CH, _D), jnp.float32),
                   pltpu.VMEM((2, _CH, _D), jnp.float32),
                   pltpu.SemaphoreType.DMA,
                   pltpu.SemaphoreType.DMA,
                   pltpu.SemaphoreType.DMA,
               ])
    def sc_kernel(emb_hbm, i_hbm, zn_hbm, o_hbm,
                  idx_v, g_v, a_v, gsem, zsem, osem):
        wid = lax.axis_index("s") * 2 + lax.axis_index("c")
        base = wid * _RPW
        pltpu.sync_copy(i_hbm.at[pl.ds(base, _RPW)], idx_v)

        def start_in(ch):
            b = ch % 2
            g = pltpu.async_copy(
                emb_hbm.at[idx_v.at[pl.ds(ch * _CH, _CH)]], g_v.at[b], gsem)
            z = pltpu.async_copy(
                zn_hbm.at[pl.ds(base + ch * _CH, _CH)], a_v.at[b], zsem)
            return g, z

        def compute(ch):
            b = ch % 2

            @pl.loop(0, _CH)
            def _(r):
                for cc in range(0, _D, 16):
                    gx = g_v.at[b, r, pl.ds(cc, 16)][...]
                    plsc.addupdate(a_v.at[b, r, pl.ds(cc, 16)], 2.0 * gx)

        def start_out(ch):
            b = ch % 2
            return pltpu.async_copy(
                a_v.at[b], o_hbm.at[pl.ds(base + ch * _CH, _CH)], osem)

        ins = [start_in(0), start_in(1)]
        outs = []
        for ch in range(_NCH):
            g, z = ins[ch]
            g.wait()
            z.wait()
            compute(ch)
            outs.append(start_out(ch))
            if ch + 2 < _NCH:
                outs[ch].wait()
                ins.append(start_in(ch + 2))
        outs[_NCH - 2].wait()
        outs[_NCH - 1].wait()

    return sc_kernel(emb, idx, zneg)


def kernel(z, emb):
    z_flat = z.reshape(_N, _D)
    idx, zneg = _argmin_call(z_flat, emb)
    out = _gather_combine(emb, idx, zneg)
    return out.reshape(z.shape)


# speedup vs baseline: 1.0733x; 1.0733x over previous
import jax
import jax.numpy as jnp
from jax import lax
from jax.experimental import pallas as pl
from jax.experimental.pallas import tpu as pltpu
from jax.experimental.pallas import tpu_sc as plsc

_N = 8192
_K = 8192
_D = 256
_NB = 4096
_KB = 4096
_GN = _N // _NB
_GK = _K // _KB


def _argmin_body(z_ref, e_ref, idx_ref, zneg_ref,
                 nrm_s, best_s, zbf_s, ebf_s):
    n = pl.program_id(0)
    k = pl.program_id(1)

    @pl.when(k == 0)
    def _():
        zf = z_ref[...]
        zbf_s[...] = zf.astype(jnp.bfloat16)
        zneg_ref[...] = -zf

    @pl.when(n == 0)
    def _():
        ef = e_ref[...]
        ebf_s[k] = ef.astype(jnp.bfloat16)
        nrm_s[k] = jnp.reshape(1.0 - 0.5 * jnp.sum(ef * ef, axis=1),
                               (_KB // 128, 128))

    idc = (_GK - 1 - k) << 5
    nrm_rows = nrm_s[k]
    zbf = zbf_s[...]
    bv = None
    for g in range(_KB // 256):
        sg = lax.dot_general(
            zbf, ebf_s[k, g * 256:(g + 1) * 256, :],
            (((1,), (1,)), ((), ())),
            preferred_element_type=jnp.float32)
        for jj in range(2):
            j = 2 * g + jj
            sl = slice(jj * 128, (jj + 1) * 128)
            sj = sg[:, sl] + nrm_rows[j:j + 1, :]
            kj = ((lax.bitcast_convert_type(sj, jnp.int32) & jnp.int32(-64))
                  | (idc | (_KB // 128 - 1 - j)))
            kjf = lax.bitcast_convert_type(kj, jnp.float32)
            bv = kjf if bv is None else jnp.maximum(bv, kjf)

    @pl.when(k == 0)
    def _():
        best_s[...] = bv

    @pl.when(k > 0)
    def _():
        best_s[...] = jnp.maximum(best_s[...], bv)

    @pl.when(k == _GK - 1)
    def _():
        b128 = best_s[...]
        kmax = jnp.max(b128, axis=1, keepdims=True)
        lane_iota = lax.broadcasted_iota(jnp.int32, (_NB, 128), 1)
        lane = jnp.min(jnp.where(b128 == kmax, lane_iota, jnp.int32(2 ** 30)),
                       axis=1, keepdims=True)
        id6 = lax.bitcast_convert_type(kmax, jnp.int32) & 63
        ktile = _GK - 1 - lax.shift_right_logical(id6, 5)
        strip = _KB // 128 - 1 - (id6 & 31)
        b = ktile * _KB + strip * 128 + lane
        idx_ref[...] = jnp.reshape(b, (_NB,))


_argmin_call = pl.pallas_call(
    _argmin_body,
    grid=(_GN, _GK),
    in_specs=[
        pl.BlockSpec((_NB, _D), lambda n, k: (n, 0)),
        pl.BlockSpec((_KB, _D), lambda n, k: (jnp.where(n == 0, k, _GK - 1), 0)),
    ],
    out_specs=[
        pl.BlockSpec((_NB,), lambda n, k: (n,)),
        pl.BlockSpec((_NB, _D), lambda n, k: (n, 0)),
    ],
    out_shape=[
        jax.ShapeDtypeStruct((_N,), jnp.int32),
        jax.ShapeDtypeStruct((_N, _D), jnp.float32),
    ],
    scratch_shapes=[
        pltpu.VMEM((_GK, _KB // 128, 128), jnp.float32),
        pltpu.VMEM((_NB, 128), jnp.float32),
        pltpu.VMEM((_NB, _D), jnp.bfloat16),
        pltpu.VMEM((_GK, _KB, _D), jnp.bfloat16),
    ],
    compiler_params=pltpu.CompilerParams(
        dimension_semantics=("arbitrary", "arbitrary")),
)

_CH = 64
_RPW = _N // 32
_NCH = _RPW // _CH


def _gather_combine(emb, idx, zneg):
    mesh = plsc.VectorSubcoreMesh(core_axis_name="c", subcore_axis_name="s")

    @pl.kernel(out_type=jax.ShapeDtypeStruct((_N, _D), jnp.float32),
               mesh=mesh,
               scratch_types=[
                   pltpu.VMEM((_RPW,), jnp.int32),
                   pltpu.VMEM((2, _CH, _D), jnp.float32),
                   pltpu.VMEM((2, _CH, _D), jnp.float32),
                   pltpu.SemaphoreType.DMA,
                   pltpu.SemaphoreType.DMA,
                   pltpu.SemaphoreType.DMA,
               ])
    def sc_kernel(emb_hbm, i_hbm, zn_hbm, o_hbm,
                  idx_v, g_v, a_v, gsem, zsem, osem):
        wid = lax.axis_index("s") * 2 + lax.axis_index("c")
        base = wid * _RPW
        pltpu.sync_copy(i_hbm.at[pl.ds(base, _RPW)], idx_v)

        def start_in(ch):
            b = ch % 2
            g = pltpu.async_copy(
                emb_hbm.at[idx_v.at[pl.ds(ch * _CH, _CH)]], g_v.at[b], gsem)
            z = pltpu.async_copy(
                zn_hbm.at[pl.ds(base + ch * _CH, _CH)], a_v.at[b], zsem)
            return g, z

        def compute(ch):
            b = ch % 2

            @pl.loop(0, _CH)
            def _(r):
                for cc in range(0, _D, 16):
                    gx = g_v.at[b, r, pl.ds(cc, 16)][...]
                    plsc.addupdate(a_v.at[b, r, pl.ds(cc, 16)], 2.0 * gx)

        def start_out(ch):
            b = ch % 2
            return pltpu.async_copy(
                a_v.at[b], o_hbm.at[pl.ds(base + ch * _CH, _CH)], osem)

        ins = [start_in(0), start_in(1)]
        outs = []
        for ch in range(_NCH):
            g, z = ins[ch]
            g.wait()
            z.wait()
            compute(ch)
            outs.append(start_out(ch))
            if ch + 2 < _NCH:
                outs[ch].wait()
                ins.append(start_in(ch + 2))
        outs[_NCH - 2].wait()
        outs[_NCH - 1].wait()

    return sc_kernel(emb, idx, zneg)


def kernel(z, emb):
    z_flat = z.reshape(_N, _D)
    idx, zneg = _argmin_call(z_flat, emb)
    out = _gather_combine(emb, idx, zneg)
    return out.reshape(z.shape)
